# baseline (speedup 1.0000x reference)
import functools

import jax
import jax.numpy as jnp
from jax import lax
from jax.experimental import pallas as pl
from jax.experimental.pallas import tpu as pltpu
from jax.experimental.pallas import tpu_sc as plsc

B = 4
S = 2048
D = 768
NC = 2
NS = 16
NW = NC * NS
P = S // NW
C = 32
H = P // C
NCH = B * H
LANES = 16
NCOL = D // LANES


NBUF = 3


def _body(tok_hbm, emb_hbm, pos_hbm, out_hbm,
          pos_v, idx_all, buf0, buf1, buf2,
          psem, gsem0, gsem1, gsem2, osem0, osem1, osem2):
    wid = lax.axis_index("s") * NC + lax.axis_index("c")
    p0 = wid * P

    bufs = (buf0, buf1, buf2)
    gsems = (gsem0, gsem1, gsem2)
    osems = (osem0, osem1, osem2)

    pos_cp = pltpu.async_copy(pos_hbm.at[pl.ds(p0, P)], pos_v, psem)
    for b in range(B):
        pltpu.sync_copy(tok_hbm.at[b, pl.ds(p0, P)], idx_all.at[b])

    def start_gather(i):
        b, h = divmod(i, H)
        s = i % NBUF
        return pltpu.async_copy(
            emb_hbm.at[idx_all.at[b, pl.ds(h * C, C)]], bufs[s], gsems[s])

    gat_cp = [start_gather(0), start_gather(1), None]
    out_cp = [None, None, None]
    pos_cp.wait()

    for i in range(NCH):
        s = i % NBUF
        b, h = divmod(i, H)
        g = i + 2
        if g < NCH:
            if g >= NBUF:
                out_cp[g % NBUF].wait()
            gat_cp[g % NBUF] = start_gather(g)
        gat_cp[s].wait()
        buf = bufs[s]

        @plsc.parallel_loop(0, C, 1, unroll=4)
        def _(r, buf=buf, h=h):
            pr = h * C + r
            for c in range(NCOL):
                sl = pl.ds(c * LANES, LANES)
                buf[r, sl] = buf[r, sl] + pos_v[pr, sl]

        out_cp[s] = pltpu.async_copy(
            buf, out_hbm.at[b, pl.ds(p0 + h * C, C)], osems[s])

    for i in range(NCH - NBUF, NCH):
        out_cp[i % NBUF].wait()


@functools.partial(jax.jit, static_argnames=())
def _run(tok_flat, emb, pos_emb):
    mesh = plsc.VectorSubcoreMesh(core_axis_name="c", subcore_axis_name="s")
    f = pl.kernel(
        _body,
        out_type=jax.ShapeDtypeStruct((B, S, D), jnp.float32),
        mesh=mesh,
        scratch_types=[
            pltpu.VMEM((P, D), jnp.float32),
            pltpu.VMEM((B, P), jnp.int32),
            pltpu.VMEM((C, D), jnp.float32),
            pltpu.VMEM((C, D), jnp.float32),
            pltpu.VMEM((C, D), jnp.float32),
            pltpu.SemaphoreType.DMA,
            pltpu.SemaphoreType.DMA,
            pltpu.SemaphoreType.DMA,
            pltpu.SemaphoreType.DMA,
            pltpu.SemaphoreType.DMA,
            pltpu.SemaphoreType.DMA,
            pltpu.SemaphoreType.DMA,
        ],
    )
    return f(tok_flat, emb, pos_emb)


def kernel(tokens, emb, pos_emb):
    return _run(tokens.astype(jnp.int32), emb, pos_emb)


# speedup vs baseline: 1.0673x; 1.0673x over previous
import functools

import jax
import jax.numpy as jnp
from jax import lax
from jax.experimental import pallas as pl
from jax.experimental.pallas import tpu as pltpu
from jax.experimental.pallas import tpu_sc as plsc

B = 4
S = 2048
D = 768
NC = 2
NS = 16
NW = NC * NS
P = S // NW
C = 32
H = P // C
NCH = B * H
LANES = 16
NCOL = D // LANES


NBUF = 3


def _body(tok_hbm, emb_hbm, pos_hbm, out_hbm,
          pos_v, idx_all, buf0, buf1, buf2,
          psem, gsem0, gsem1, gsem2, osem0, osem1, osem2):
    wid = lax.axis_index("s") * NC + lax.axis_index("c")
    p0 = wid * P

    bufs = (buf0, buf1, buf2)
    gsems = (gsem0, gsem1, gsem2)
    osems = (osem0, osem1, osem2)

    pos_cp = pltpu.async_copy(pos_hbm.at[pl.ds(p0, P)], pos_v, psem)
    for b in range(B):
        pltpu.sync_copy(tok_hbm.at[b, pl.ds(p0, P)], idx_all.at[b])

    def start_gather(i):
        b, h = divmod(i, H)
        s = i % NBUF
        return pltpu.async_copy(
            emb_hbm.at[idx_all.at[b, pl.ds(h * C, C)]], bufs[s], gsems[s])

    gat_cp = [start_gather(0), start_gather(1), None]
    out_cp = [None, None, None]
    pos_cp.wait()

    for i in range(NCH):
        s = i % NBUF
        b, h = divmod(i, H)
        g = i + 2
        if g < NCH:
            if g >= NBUF:
                out_cp[g % NBUF].wait()
            gat_cp[g % NBUF] = start_gather(g)
        gat_cp[s].wait()
        buf = bufs[s]

        @plsc.parallel_loop(0, C, 1, unroll=2)
        def _(r, buf=buf, h=h):
            pr = h * C + r
            for c in range(NCOL):
                sl = pl.ds(c * LANES, LANES)
                buf[r, sl] = buf[r, sl] + pos_v[pr, sl]

        out_cp[s] = pltpu.async_copy(
            buf, out_hbm.at[b, pl.ds(p0 + h * C, C)], osems[s])

    for i in range(NCH - NBUF, NCH):
        out_cp[i % NBUF].wait()


@functools.partial(jax.jit, static_argnames=())
def _run(tok_flat, emb, pos_emb):
    mesh = plsc.VectorSubcoreMesh(core_axis_name="c", subcore_axis_name="s")
    f = pl.kernel(
        _body,
        out_type=jax.ShapeDtypeStruct((B, S, D), jnp.float32),
        mesh=mesh,
        scratch_types=[
            pltpu.VMEM((P, D), jnp.float32),
            pltpu.VMEM((B, P), jnp.int32),
            pltpu.VMEM((C, D), jnp.float32),
            pltpu.VMEM((C, D), jnp.float32),
            pltpu.VMEM((C, D), jnp.float32),
            pltpu.SemaphoreType.DMA,
            pltpu.SemaphoreType.DMA,
            pltpu.SemaphoreType.DMA,
            pltpu.SemaphoreType.DMA,
            pltpu.SemaphoreType.DMA,
            pltpu.SemaphoreType.DMA,
            pltpu.SemaphoreType.DMA,
        ],
    )
    return f(tok_flat, emb, pos_emb)


def kernel(tokens, emb, pos_emb):
    return _run(tokens.astype(jnp.int32), emb, pos_emb)


# speedup vs baseline: 1.2027x; 1.1269x over previous
import jax
import jax.numpy as jnp
from jax import lax
from jax.experimental import pallas as pl
from jax.experimental.pallas import tpu as pltpu
from jax.experimental.pallas import tpu_sc as plsc

B = 4
S = 2048
D = 768
NC = 2
NS = 16
NW = NC * NS
P = S // NW
C = 16
G = P // C
LANES = 16
NCOL = D // LANES


def _body(tok_hbm, emb_hbm, pos_hbm, out_hbm,
          idx_all, pos0, pos1,
          b00, b01, b02, b03, b10, b11, b12, b13,
          ps0, ps1, gs0, gs1, os0, os1):
    wid = lax.axis_index("s") * NC + lax.axis_index("c")
    p0 = wid * P

    poss = (pos0, pos1)
    bufsets = ((b00, b01, b02, b03), (b10, b11, b12, b13))
    psems = (ps0, ps1)
    gsems = (gs0, gs1)
    osems = (os0, os1)

    for b in range(B):
        pltpu.sync_copy(tok_hbm.at[b, pl.ds(p0, P)], idx_all.at[b])

    def start_group(g):
        s = g % 2
        pcp = pltpu.async_copy(
            pos_hbm.at[pl.ds(p0 + g * C, C)], poss[s], psems[s])
        gcps = [
            pltpu.async_copy(
                emb_hbm.at[idx_all.at[b, pl.ds(g * C, C)]],
                bufsets[s][b], gsems[s])
            for b in range(B)
        ]
        return pcp, gcps

    grp = [start_group(0), start_group(1), None, None]
    outs = [None] * G

    for g in range(G):
        s = g % 2
        pcp, gcps = grp[g]
        pcp.wait()
        for q in range(B):
            gcps[q].wait()
        pos_s = poss[s]
        bset = bufsets[s]

        @plsc.parallel_loop(0, C, 1, unroll=2)
        def _(r, pos_s=pos_s, bset=bset):
            for c in range(NCOL):
                sl = pl.ds(c * LANES, LANES)
                pv = pos_s[r, sl]
                for q in range(B):
                    bset[q][r, sl] = bset[q][r, sl] + pv

        outs[g] = [
            pltpu.async_copy(
                bset[q], out_hbm.at[q, pl.ds(p0 + g * C, C)], osems[s])
            for q in range(B)
        ]
        if g + 2 < G:
            for cp in outs[g]:
                cp.wait()
            grp[g + 2] = start_group(g + 2)

    for g in (G - 2, G - 1):
        for cp in outs[g]:
            cp.wait()


@jax.jit
def _run(tokens, emb, pos_emb):
    mesh = plsc.VectorSubcoreMesh(core_axis_name="c", subcore_axis_name="s")
    buf = pltpu.VMEM((C, D), jnp.float32)
    f = pl.kernel(
        _body,
        out_type=jax.ShapeDtypeStruct((B, S, D), jnp.float32),
        mesh=mesh,
        scratch_types=[
            pltpu.VMEM((B, P), jnp.int32),
            buf, buf,
            buf, buf, buf, buf,
            buf, buf, buf, buf,
            pltpu.SemaphoreType.DMA,
            pltpu.SemaphoreType.DMA,
            pltpu.SemaphoreType.DMA,
            pltpu.SemaphoreType.DMA,
            pltpu.SemaphoreType.DMA,
            pltpu.SemaphoreType.DMA,
        ],
    )
    return f(tokens, emb, pos_emb)


def kernel(tokens, emb, pos_emb):
    return _run(tokens.astype(jnp.int32), emb, pos_emb)
